# baseline (speedup 1.0000x reference)
import functools

import jax
import jax.numpy as jnp
from jax import lax
from jax.experimental import pallas as pl
from jax.experimental.pallas import tpu as pltpu
from jax.experimental.pallas import tpu_sc as plsc

_GRID = 256
_HW = _GRID * _GRID
_HALF = _HW // 2
_PIECE = 4096


@functools.lru_cache(maxsize=None)
def _build(B, N):
    ROWS_PER_B = N // 64
    CHROWS = 64
    NCHUNK = (ROWS_PER_B // 2) // CHROWS
    assert NCHUNK % 2 == 0
    UNROLL = 8

    mesh = plsc.VectorSubcoreMesh(core_axis_name="c", subcore_axis_name="s")

    @functools.partial(
        pl.kernel,
        mesh=mesh,
        out_type=[
            jax.ShapeDtypeStruct((B * ROWS_PER_B, 128), jnp.int32),
            jax.ShapeDtypeStruct((B * _HW // 128, 128), jnp.int32),
            jax.ShapeDtypeStruct((32, _HALF), jnp.int32),
        ],
        scratch_types=[
            pltpu.VMEM((2, CHROWS, 128), jnp.float32),
            pltpu.VMEM((2, CHROWS, 128), jnp.int32),
            pltpu.VMEM((_HW,), jnp.int32),
            pltpu.VMEM((2, _PIECE), jnp.int32),
            pltpu.VMEM((2, 32, 128), jnp.int32),
            pltpu.SemaphoreType.DMA,
            pltpu.SemaphoreType.DMA,
            pltpu.SemaphoreType.DMA,
            pltpu.SemaphoreType.DMA,
        ],
        compiler_params=pltpu.CompilerParams(needs_layout_passes=False),
    )
    def _k(xy_hbm, q_hbm, vox_hbm, scr_hbm, xybuf, qbuf, hist, mbuf, stag,
           sem_in0, sem_in1, sem_out0, sem_out1):
        c = lax.axis_index("c")
        s = lax.axis_index("s")
        h = s % 2
        b = c * (B // 2) + s // 2
        w = c * 16 + s

        sem_in = (sem_in0, sem_in1)
        sem_out = (sem_out0, sem_out1)
        row0 = b * ROWS_PER_B + h * (ROWS_PER_B // 2)

        def in_copy(ci, k):
            return pltpu.make_async_copy(
                xy_hbm.at[pl.ds(row0 + ci * CHROWS, CHROWS), :],
                xybuf.at[k], sem_in[k])

        def out_copy(ci, k):
            return pltpu.make_async_copy(
                qbuf.at[k],
                q_hbm.at[pl.ds(row0 + ci * CHROWS, CHROWS), :],
                sem_out[k])

        ones = jnp.full((16,), 1, jnp.int32)
        zeros = jnp.zeros((16,), jnp.int32)

        in_copy(0, 0).start()

        @plsc.parallel_loop(0, _HW // (16 * 2 * UNROLL), unroll=2)
        def _zero(i):
            for u in range(2 * UNROLL):
                hist[pl.ds((i * 2 * UNROLL + u) * 16, 16)] = zeros

        def pair(gi, _):
            for k in range(2):
                ci = gi * 2 + k
                in_copy(ci, k).wait()

                @pl.when(ci + 1 < NCHUNK)
                def _():
                    in_copy(ci + 1, k ^ 1).start()

                @pl.when(gi >= 1)
                def _():
                    out_copy(ci - 2, k).wait()

                @plsc.parallel_loop(0, CHROWS // 2, unroll=4)
                def _points(t):
                    xr = 2 * t
                    yr = 2 * t + 1
                    for g in range(8):
                        sl = pl.ds(g * 16, 16)
                        qx = (xybuf[k, xr, sl] * float(_GRID - 1)).astype(
                            jnp.int32)
                        qy = (xybuf[k, yr, sl] * float(_GRID - 1)).astype(
                            jnp.int32)
                        qbuf[k, xr, sl] = qx
                        qbuf[k, yr, sl] = qy
                        plsc.addupdate_scatter(hist, [qx + (qy << 8)], ones)

                out_copy(ci, k).start()
            return 0

        lax.fori_loop(0, NCHUNK // 2, pair, 0)
        out_copy(NCHUNK - 2, 0).wait()
        out_copy(NCHUNK - 1, 1).wait()

        oh = (1 - h) * _HALF
        mh = h * _HALF
        vrow0 = b * (_HW // 128) + h * (_HALF // 128)
        pltpu.sync_copy(hist.at[pl.ds(oh, _HALF)], scr_hbm.at[w])
        plsc.subcore_barrier()

        NPIECE = _HALF // _PIECE

        def piece_in(p, k):
            return pltpu.make_async_copy(
                scr_hbm.at[c * 16 + (s ^ 1), pl.ds(p * _PIECE, _PIECE)],
                mbuf.at[k], sem_in[k])

        def vout_copy(P, kp):
            return pltpu.make_async_copy(
                stag.at[kp],
                vox_hbm.at[pl.ds(vrow0 + P * 32, 32), :], sem_out[kp])

        piece_in(0, 0).start()

        def mpair(gp, _):
            for kp in range(2):
                p = gp * 2 + kp
                piece_in(p, kp).wait()

                @pl.when(p + 1 < NPIECE)
                def _():
                    piece_in(p + 1, kp ^ 1).start()

                @pl.when(gp >= 1)
                def _():
                    vout_copy(p - 2, kp).wait()

                @plsc.parallel_loop(0, 32, unroll=2)
                def _madd(i):
                    row = (i >> 4) * 16 + (i & 1) * 8 + ((i >> 1) & 7)
                    for u in range(UNROLL):
                        sl = pl.ds(u * 16, 16)
                        src = mh + p * _PIECE + i * 128 + u * 16
                        stag[kp, row, sl] = (
                            hist[pl.ds(src, 16)]
                            + mbuf[kp, pl.ds(i * 128 + u * 16, 16)])

                vout_copy(p, kp).start()
            return 0

        lax.fori_loop(0, NPIECE // 2, mpair, 0)
        vout_copy(NPIECE - 2, 0).wait()
        vout_copy(NPIECE - 1, 1).wait()

    return _k


def kernel(xy):
    B, N, _ = xy.shape
    xt = (xy.reshape(B, N // 128, 128, 2)
          .transpose(0, 1, 3, 2)
          .reshape(B * N // 64, 128))
    q_flat, vox_flat, _scr = _build(B, N)(xt)
    q = (q_flat.reshape(B, N // 128, 2, 128)
         .transpose(0, 1, 3, 2)
         .reshape(B, N, 2))
    vox = (vox_flat.reshape(B, _GRID // 8, 2, 8, 128)
           .transpose(0, 1, 3, 2, 4)
           .reshape(B, _GRID, _GRID))
    return q, vox


# speedup vs baseline: 1.0146x; 1.0146x over previous
import functools

import jax
import jax.numpy as jnp
from jax import lax
from jax.experimental import pallas as pl
from jax.experimental.pallas import tpu as pltpu
from jax.experimental.pallas import tpu_sc as plsc

_GRID = 256
_HW = _GRID * _GRID
_HALF = _HW // 2
_PIECE = 4096


@functools.lru_cache(maxsize=None)
def _build(B, N):
    ROWS_PER_B = N // 64
    CHROWS = 64
    NCHUNK = (ROWS_PER_B // 2) // CHROWS
    assert NCHUNK % 2 == 0
    UNROLL = 8

    mesh = plsc.VectorSubcoreMesh(core_axis_name="c", subcore_axis_name="s")

    @functools.partial(
        pl.kernel,
        mesh=mesh,
        out_type=[
            jax.ShapeDtypeStruct((B * ROWS_PER_B, 128), jnp.int32),
            jax.ShapeDtypeStruct((B * _HW // 128, 128), jnp.int32),
            jax.ShapeDtypeStruct((32, _HALF), jnp.int32),
        ],
        scratch_types=[
            pltpu.VMEM((2, CHROWS, 128), jnp.float32),
            pltpu.VMEM((2, CHROWS, 128), jnp.int32),
            pltpu.VMEM((_HW,), jnp.int32),
            pltpu.VMEM((2, _PIECE), jnp.int32),
            pltpu.VMEM((2, 32, 128), jnp.int32),
            pltpu.SemaphoreType.DMA,
            pltpu.SemaphoreType.DMA,
            pltpu.SemaphoreType.DMA,
            pltpu.SemaphoreType.DMA,
        ],
        compiler_params=pltpu.CompilerParams(needs_layout_passes=False),
    )
    def _k(xy_hbm, q_hbm, vox_hbm, scr_hbm, xybuf, qbuf, hist, mbuf, stag,
           sem_in0, sem_in1, sem_out0, sem_out1):
        c = lax.axis_index("c")
        s = lax.axis_index("s")
        h = s % 2
        b = c * (B // 2) + s // 2
        w = c * 16 + s

        sem_in = (sem_in0, sem_in1)
        sem_out = (sem_out0, sem_out1)
        row0 = b * ROWS_PER_B + h * (ROWS_PER_B // 2)

        def in_copy(ci, k):
            return pltpu.make_async_copy(
                xy_hbm.at[pl.ds(row0 + ci * CHROWS, CHROWS), :],
                xybuf.at[k], sem_in[k])

        def out_copy(ci, k):
            return pltpu.make_async_copy(
                qbuf.at[k],
                q_hbm.at[pl.ds(row0 + ci * CHROWS, CHROWS), :],
                sem_out[k])

        ones = jnp.full((16,), 1, jnp.int32)
        zeros = jnp.zeros((16,), jnp.int32)

        in_copy(0, 0).start()

        @plsc.parallel_loop(0, _HW // (16 * 2 * UNROLL), unroll=2)
        def _zero(i):
            for u in range(2 * UNROLL):
                hist[pl.ds((i * 2 * UNROLL + u) * 16, 16)] = zeros

        def pair(gi, _):
            for k in range(2):
                ci = gi * 2 + k
                in_copy(ci, k).wait()

                @pl.when(ci + 1 < NCHUNK)
                def _():
                    in_copy(ci + 1, k ^ 1).start()

                @pl.when(gi >= 1)
                def _():
                    out_copy(ci - 2, k).wait()

                @plsc.parallel_loop(0, CHROWS // 2, unroll=2)
                def _points(t):
                    xr = 2 * t
                    yr = 2 * t + 1
                    for g in range(8):
                        sl = pl.ds(g * 16, 16)
                        qx = (xybuf[k, xr, sl] * float(_GRID - 1)).astype(
                            jnp.int32)
                        qy = (xybuf[k, yr, sl] * float(_GRID - 1)).astype(
                            jnp.int32)
                        qbuf[k, xr, sl] = qx
                        qbuf[k, yr, sl] = qy
                        plsc.addupdate_scatter(hist, [qx + (qy << 8)], ones)

                out_copy(ci, k).start()
            return 0

        lax.fori_loop(0, NCHUNK // 2, pair, 0)
        out_copy(NCHUNK - 2, 0).wait()
        out_copy(NCHUNK - 1, 1).wait()

        oh = (1 - h) * _HALF
        mh = h * _HALF
        vrow0 = b * (_HW // 128) + h * (_HALF // 128)
        pltpu.sync_copy(hist.at[pl.ds(oh, _HALF)], scr_hbm.at[w])
        plsc.subcore_barrier()

        NPIECE = _HALF // _PIECE

        def piece_in(p, k):
            return pltpu.make_async_copy(
                scr_hbm.at[c * 16 + (s ^ 1), pl.ds(p * _PIECE, _PIECE)],
                mbuf.at[k], sem_in[k])

        def vout_copy(P, kp):
            return pltpu.make_async_copy(
                stag.at[kp],
                vox_hbm.at[pl.ds(vrow0 + P * 32, 32), :], sem_out[kp])

        piece_in(0, 0).start()

        def mpair(gp, _):
            for kp in range(2):
                p = gp * 2 + kp
                piece_in(p, kp).wait()

                @pl.when(p + 1 < NPIECE)
                def _():
                    piece_in(p + 1, kp ^ 1).start()

                @pl.when(gp >= 1)
                def _():
                    vout_copy(p - 2, kp).wait()

                @plsc.parallel_loop(0, 32, unroll=2)
                def _madd(i):
                    row = (i >> 4) * 16 + (i & 1) * 8 + ((i >> 1) & 7)
                    for u in range(UNROLL):
                        sl = pl.ds(u * 16, 16)
                        src = mh + p * _PIECE + i * 128 + u * 16
                        stag[kp, row, sl] = (
                            hist[pl.ds(src, 16)]
                            + mbuf[kp, pl.ds(i * 128 + u * 16, 16)])

                vout_copy(p, kp).start()
            return 0

        lax.fori_loop(0, NPIECE // 2, mpair, 0)
        vout_copy(NPIECE - 2, 0).wait()
        vout_copy(NPIECE - 1, 1).wait()

    return _k


def kernel(xy):
    B, N, _ = xy.shape
    xt = (xy.reshape(B, N // 128, 128, 2)
          .transpose(0, 1, 3, 2)
          .reshape(B * N // 64, 128))
    q_flat, vox_flat, _scr = _build(B, N)(xt)
    q = (q_flat.reshape(B, N // 128, 2, 128)
         .transpose(0, 1, 3, 2)
         .reshape(B, N, 2))
    vox = (vox_flat.reshape(B, _GRID // 8, 2, 8, 128)
           .transpose(0, 1, 3, 2, 4)
           .reshape(B, _GRID, _GRID))
    return q, vox


# speedup vs baseline: 1.0214x; 1.0067x over previous
import functools

import jax
import jax.numpy as jnp
from jax import lax
from jax.experimental import pallas as pl
from jax.experimental.pallas import tpu as pltpu
from jax.experimental.pallas import tpu_sc as plsc

_GRID = 256
_HW = _GRID * _GRID
_HALF = _HW // 2
_PIECE = 4096


@functools.lru_cache(maxsize=None)
def _build(B, N):
    ROWS_PER_B = N // 64
    CHROWS = 64
    NCHUNK = (ROWS_PER_B // 2) // CHROWS
    assert NCHUNK % 2 == 0
    UNROLL = 8

    mesh = plsc.VectorSubcoreMesh(core_axis_name="c", subcore_axis_name="s")

    @functools.partial(
        pl.kernel,
        mesh=mesh,
        out_type=[
            jax.ShapeDtypeStruct((B * ROWS_PER_B, 128), jnp.int32),
            jax.ShapeDtypeStruct((B * _HW // 128, 128), jnp.int32),
            jax.ShapeDtypeStruct((32, _HALF), jnp.int32),
        ],
        scratch_types=[
            pltpu.VMEM((2, CHROWS, 128), jnp.float32),
            pltpu.VMEM((2, CHROWS, 128), jnp.int32),
            pltpu.VMEM((_HW,), jnp.int32),
            pltpu.VMEM((2, _PIECE), jnp.int32),
            pltpu.VMEM((2, 32, 128), jnp.int32),
            pltpu.SemaphoreType.DMA,
            pltpu.SemaphoreType.DMA,
            pltpu.SemaphoreType.DMA,
            pltpu.SemaphoreType.DMA,
        ],
        compiler_params=pltpu.CompilerParams(needs_layout_passes=False),
    )
    def _k(xy_hbm, q_hbm, vox_hbm, scr_hbm, xybuf, qbuf, hist, mbuf, stag,
           sem_in0, sem_in1, sem_out0, sem_out1):
        c = lax.axis_index("c")
        s = lax.axis_index("s")
        h = s % 2
        b = c * (B // 2) + s // 2
        w = c * 16 + s

        sem_in = (sem_in0, sem_in1)
        sem_out = (sem_out0, sem_out1)
        row0 = b * ROWS_PER_B + h * (ROWS_PER_B // 2)

        def in_copy(ci, k):
            return pltpu.make_async_copy(
                xy_hbm.at[pl.ds(row0 + ci * CHROWS, CHROWS), :],
                xybuf.at[k], sem_in[k])

        def out_copy(ci, k):
            return pltpu.make_async_copy(
                qbuf.at[k],
                q_hbm.at[pl.ds(row0 + ci * CHROWS, CHROWS), :],
                sem_out[k])

        ones = jnp.full((16,), 1, jnp.int32)
        zeros = jnp.zeros((16,), jnp.int32)

        in_copy(0, 0).start()

        @plsc.parallel_loop(0, _HW // (16 * 2 * UNROLL), unroll=2)
        def _zero(i):
            for u in range(2 * UNROLL):
                hist[pl.ds((i * 2 * UNROLL + u) * 16, 16)] = zeros

        def pair(gi, _):
            for k in range(2):
                ci = gi * 2 + k
                in_copy(ci, k).wait()

                @pl.when(ci + 1 < NCHUNK)
                def _():
                    in_copy(ci + 1, k ^ 1).start()

                @pl.when(gi >= 1)
                def _():
                    out_copy(ci - 2, k).wait()

                @plsc.parallel_loop(0, CHROWS // 2, unroll=2)
                def _points(t):
                    xr = 2 * t
                    yr = 2 * t + 1
                    for g in range(8):
                        sl = pl.ds(g * 16, 16)
                        qx = (xybuf[k, xr, sl] * float(_GRID - 1)).astype(
                            jnp.int32)
                        qy = (xybuf[k, yr, sl] * float(_GRID - 1)).astype(
                            jnp.int32)
                        qbuf[k, xr, sl] = qx
                        qbuf[k, yr, sl] = qy
                        plsc.addupdate_scatter(hist, [qx + (qy << 8)], ones)

                out_copy(ci, k).start()
            return 0

        lax.fori_loop(0, NCHUNK // 2, pair, 0)

        oh = (1 - h) * _HALF
        mh = h * _HALF
        vrow0 = b * (_HW // 128) + h * (_HALF // 128)
        pltpu.sync_copy(hist.at[pl.ds(oh, _HALF)], scr_hbm.at[w])
        out_copy(NCHUNK - 2, 0).wait()
        out_copy(NCHUNK - 1, 1).wait()
        plsc.subcore_barrier()

        NPIECE = _HALF // _PIECE

        def piece_in(p, k):
            return pltpu.make_async_copy(
                scr_hbm.at[c * 16 + (s ^ 1), pl.ds(p * _PIECE, _PIECE)],
                mbuf.at[k], sem_in[k])

        def vout_copy(P, kp):
            return pltpu.make_async_copy(
                stag.at[kp],
                vox_hbm.at[pl.ds(vrow0 + P * 32, 32), :], sem_out[kp])

        piece_in(0, 0).start()

        def mpair(gp, _):
            for kp in range(2):
                p = gp * 2 + kp
                piece_in(p, kp).wait()

                @pl.when(p + 1 < NPIECE)
                def _():
                    piece_in(p + 1, kp ^ 1).start()

                @pl.when(gp >= 1)
                def _():
                    vout_copy(p - 2, kp).wait()

                @plsc.parallel_loop(0, 32, unroll=2)
                def _madd(i):
                    row = (i >> 4) * 16 + (i & 1) * 8 + ((i >> 1) & 7)
                    for u in range(UNROLL):
                        sl = pl.ds(u * 16, 16)
                        src = mh + p * _PIECE + i * 128 + u * 16
                        stag[kp, row, sl] = (
                            hist[pl.ds(src, 16)]
                            + mbuf[kp, pl.ds(i * 128 + u * 16, 16)])

                vout_copy(p, kp).start()
            return 0

        lax.fori_loop(0, NPIECE // 2, mpair, 0)
        vout_copy(NPIECE - 2, 0).wait()
        vout_copy(NPIECE - 1, 1).wait()

    return _k


def kernel(xy):
    B, N, _ = xy.shape
    xt = (xy.reshape(B, N // 128, 128, 2)
          .transpose(0, 1, 3, 2)
          .reshape(B * N // 64, 128))
    q_flat, vox_flat, _scr = _build(B, N)(xt)
    q = (q_flat.reshape(B, N // 128, 2, 128)
         .transpose(0, 1, 3, 2)
         .reshape(B, N, 2))
    vox = (vox_flat.reshape(B, _GRID // 8, 2, 8, 128)
           .transpose(0, 1, 3, 2, 4)
           .reshape(B, _GRID, _GRID))
    return q, vox
